# baseline (speedup 1.0000x reference)
import functools

import numpy as np
import jax
import jax.numpy as jnp
from jax import lax
from jax.experimental import pallas as pl
from jax.experimental.pallas import tpu as pltpu
from jax.experimental.pallas import tpu_sc as plsc

N = 10000
NPAD = 10112
ROWS_PER_TILE = NPAD // 16
E = 320000
NW = 32
CHUNK = 64
NCH = 160
CHUNK2 = 128
NCH2 = 80
NCH_S = 62
NCH_F = 252
NCH_PAIR = NCH_S + NCH_F
TOTCH = 16 * NCH_PAIR + NCH_F
NCH2_S = 68
NCH2_F = 90
NCH2_PAIR = NCH2_S + NCH2_F
TOTCH2 = 16 * NCH2_PAIR + NCH2_F
EPT = NCH * CHUNK
ETOT = EPT * NW
NIDS = 2000
NIDS_PAD = 2048
IDS_PER_TILE = NIDS_PAD // NW

_MESH = plsc.VectorSubcoreMesh(core_axis_name="c", subcore_axis_name="s")
_SC_PARAMS = pltpu.CompilerParams(use_tc_tiling_on_sc=False,
                                  needs_layout_passes=False)


def _tile_ids():
    c = lax.axis_index("c")
    s = lax.axis_index("s")
    return c, s, s * 2 + c


@functools.partial(
    pl.kernel,
    out_type=jax.ShapeDtypeStruct((2, NPAD, 16), jnp.float32),
    mesh=_MESH,
    scratch_types=[
        pltpu.VMEM((NCH2_F, CHUNK2), jnp.int32),
        pltpu.VMEM((CHUNK2, 16), jnp.float32),
        pltpu.VMEM_SHARED((NPAD, 16), jnp.float32),
        pltpu.SemaphoreType.DMA,
    ],
    compiler_params=_SC_PARAMS,
)
def _sc_deg(dsts_hbm, ones_hbm, zeros_hbm, out_hbm, didx, ones_v, acc, sem):
    c, s, wid = _tile_ids()
    nch = lax.select(c == 0, NCH2_S, NCH2_F)
    start = s * NCH2_PAIR + c * NCH2_S
    rows = pl.ds(s * ROWS_PER_TILE, ROWS_PER_TILE)
    pltpu.sync_copy(zeros_hbm.at[rows], acc.at[rows])
    pltpu.sync_copy(dsts_hbm.at[pl.ds(start, NCH2_F)], didx)
    pltpu.sync_copy(ones_hbm, ones_v)
    plsc.subcore_barrier()

    def fire(j, _):
        pltpu.async_copy(ones_v, acc.at[didx.at[j]], sem, add=True)
        return _

    lax.fori_loop(0, nch, fire, None)

    def drain(j, _):
        pltpu.make_async_copy(ones_v, acc.at[didx.at[j]], sem).wait()
        return _

    lax.fori_loop(0, nch, drain, None)
    plsc.subcore_barrier()
    pltpu.sync_copy(acc.at[rows], out_hbm.at[c, rows])


def _make_sc_pass(d, nch_by_core, chunk, pair):
    nch_max = max(nch_by_core)

    @functools.partial(
        pl.kernel,
        out_type=jax.ShapeDtypeStruct((2, NPAD, d), jnp.float32),
        mesh=_MESH,
        scratch_types=[
            pltpu.VMEM((nch_max, chunk), jnp.int32),
            pltpu.VMEM((nch_max, chunk), jnp.int32),
            pltpu.VMEM((2, chunk, d), jnp.float32),
            pltpu.VMEM_SHARED((NPAD, d), jnp.float32),
            pltpu.SemaphoreType.DMA,
            pltpu.SemaphoreType.DMA,
        ],
        compiler_params=_SC_PARAMS,
    )
    def _sc_pass(h_hbm, srcs_hbm, dsts_hbm, zeros_hbm, out_hbm,
                 sidx, didx, gbuf, acc, sem0, sem1):
        c, s, wid = _tile_ids()
        nch = lax.select(c == 0, nch_by_core[0], nch_by_core[1])
        start = s * pair + c * nch_by_core[0]
        pltpu.sync_copy(srcs_hbm.at[pl.ds(start, nch_max)], sidx)
        pltpu.sync_copy(dsts_hbm.at[pl.ds(start, nch_max)], didx)
        rows = pl.ds(s * ROWS_PER_TILE, ROWS_PER_TILE)
        pltpu.sync_copy(zeros_hbm.at[rows], acc.at[rows])
        plsc.subcore_barrier()

        gb0, gb1 = gbuf.at[0], gbuf.at[1]
        pltpu.async_copy(h_hbm.at[sidx.at[0]], gb0, sem0)
        pltpu.async_copy(h_hbm.at[sidx.at[1]], gb1, sem1)

        def body(j, _):
            pltpu.make_async_copy(h_hbm.at[sidx.at[2 * j]], gb0, sem0).wait()
            pltpu.sync_copy(gb0, acc.at[didx.at[2 * j]], add=True)
            pltpu.async_copy(h_hbm.at[sidx.at[2 * j + 2]], gb0, sem0)
            pltpu.make_async_copy(h_hbm.at[sidx.at[2 * j + 1]], gb1, sem1).wait()
            pltpu.sync_copy(gb1, acc.at[didx.at[2 * j + 1]], add=True)
            pltpu.async_copy(h_hbm.at[sidx.at[2 * j + 3]], gb1, sem1)
            return _

        lax.fori_loop(0, nch // 2 - 1, body, None)
        pltpu.make_async_copy(h_hbm.at[sidx.at[nch - 2]], gb0, sem0).wait()
        pltpu.sync_copy(gb0, acc.at[didx.at[nch - 2]], add=True)
        pltpu.make_async_copy(h_hbm.at[sidx.at[nch - 1]], gb1, sem1).wait()
        pltpu.sync_copy(gb1, acc.at[didx.at[nch - 1]], add=True)
        plsc.subcore_barrier()
        pltpu.sync_copy(acc.at[rows], out_hbm.at[c, rows])

    return _sc_pass


_sc_pass128 = _make_sc_pass(128, (NCH_S, NCH_F), CHUNK, NCH_PAIR)
_sc_pass16 = _make_sc_pass(16, (NCH2_S, NCH2_F), CHUNK2, NCH2_PAIR)


@functools.partial(
    pl.kernel,
    out_type=(jax.ShapeDtypeStruct((NIDS_PAD, 16), jnp.float32),
              jax.ShapeDtypeStruct((NIDS_PAD,), jnp.int32)),
    mesh=_MESH,
    scratch_types=[
        pltpu.VMEM((IDS_PER_TILE,), jnp.int32),
        pltpu.VMEM((IDS_PER_TILE, 16), jnp.float32),
        pltpu.VMEM((NPAD,), jnp.int32),
        pltpu.VMEM((IDS_PER_TILE,), jnp.int32),
        pltpu.SemaphoreType.DMA,
    ],
    compiler_params=_SC_PARAMS,
)
def _sc_pred(out2_hbm, nid_hbm, lab_hbm, yp_hbm, yt_hbm,
             nid_v, rows_v, lab_v, yt_v, sem):
    _, _, wid = _tile_ids()
    sl = pl.ds(wid * IDS_PER_TILE, IDS_PER_TILE)
    pltpu.sync_copy(nid_hbm.at[sl], nid_v)
    pltpu.sync_copy(lab_hbm, lab_v)
    pltpu.async_copy(out2_hbm.at[nid_v], rows_v, sem).wait()
    pltpu.sync_copy(rows_v, yp_hbm.at[sl])
    for k in range(IDS_PER_TILE // 16):
        idx = nid_v[pl.ds(k * 16, 16)]
        yt_v[pl.ds(k * 16, 16)] = plsc.load_gather(lab_v, [idx])
    pltpu.sync_copy(yt_v, yt_hbm.at[sl])


def _dinv_from_degp(degp_blk):
    deg = degp_blk[0, :, 0] + degp_blk[1, :, 0] + 1.0
    return lax.rsqrt(deg)


def _tc1_body(x_ref, w_ref, degp_ref, o_ref):
    dinv = _dinv_from_degp(degp_ref[...])
    h = jnp.dot(x_ref[...], w_ref[...], preferred_element_type=jnp.float32)
    o_ref[...] = h * dinv[:, None]


def _tc2_body(r1_ref, h1p_ref, degp_ref, b1_ref, w2_ref, o_ref):
    dinv = _dinv_from_degp(degp_ref[...])
    s = r1_ref[0] + r1_ref[1] + h1p_ref[...]
    h2 = jnp.maximum(s * dinv[:, None] + b1_ref[...], 0.0)
    o_ref[...] = jnp.dot(h2, w2_ref[...],
                         preferred_element_type=jnp.float32) * dinv[:, None]


def _tc3_body(r2_ref, h2p_ref, degp_ref, b2_ref, o_ref):
    dinv = _dinv_from_degp(degp_ref[...])
    s = r2_ref[0] + r2_ref[1] + h2p_ref[...]
    o_ref[...] = s * dinv[:, None] + b2_ref[...]


def _tc_loss_body(yp_ref, yt_ref, o_ref):
    yp = yp_ref[...].reshape(16, 128, 16)
    lab = yt_ref[...]
    m = jnp.max(yp, axis=2)
    lse = jnp.log(jnp.sum(jnp.exp(yp - m[:, :, None]), axis=2)) + m
    onehot = (lax.broadcasted_iota(jnp.int32, (16, 128, 16), 2)
              == lab[:, :, None])
    pick = jnp.sum(jnp.where(onehot, yp, 0.0), axis=2)
    ridx = (lax.broadcasted_iota(jnp.int32, (16, 128), 0) * 128
            + lax.broadcasted_iota(jnp.int32, (16, 128), 1))
    nll = jnp.where(ridx < NIDS, lse - pick, 0.0)
    o_ref[...] = (jnp.sum(nll) / NIDS).reshape(1, 1)


_RB = 1000


def _tc1(x, w1, degp):
    return pl.pallas_call(
        _tc1_body,
        grid=(N // _RB,),
        in_specs=[
            pl.BlockSpec((_RB, 128), lambda i: (i, 0)),
            pl.BlockSpec((128, 128), lambda i: (0, 0)),
            pl.BlockSpec((2, _RB, 16), lambda i: (0, i, 0)),
        ],
        out_specs=pl.BlockSpec((_RB, 128), lambda i: (i, 0)),
        out_shape=jax.ShapeDtypeStruct((N, 128), jnp.float32),
    )(x, w1, degp)


def _tc2(r1, h1p, degp, b1, w2):
    return pl.pallas_call(
        _tc2_body,
        grid=(N // _RB,),
        in_specs=[
            pl.BlockSpec((2, _RB, 128), lambda i: (0, i, 0)),
            pl.BlockSpec((_RB, 128), lambda i: (i, 0)),
            pl.BlockSpec((2, _RB, 16), lambda i: (0, i, 0)),
            pl.BlockSpec((1, 128), lambda i: (0, 0)),
            pl.BlockSpec((128, 16), lambda i: (0, 0)),
        ],
        out_specs=pl.BlockSpec((_RB, 16), lambda i: (i, 0)),
        out_shape=jax.ShapeDtypeStruct((N, 16), jnp.float32),
    )(r1, h1p, degp, b1, w2)


def _tc3(r2, h2p, degp, b2):
    return pl.pallas_call(
        _tc3_body,
        grid=(N // _RB,),
        in_specs=[
            pl.BlockSpec((2, _RB, 16), lambda i: (0, i, 0)),
            pl.BlockSpec((_RB, 16), lambda i: (i, 0)),
            pl.BlockSpec((2, _RB, 16), lambda i: (0, i, 0)),
            pl.BlockSpec((1, 16), lambda i: (0, 0)),
        ],
        out_specs=pl.BlockSpec((_RB, 16), lambda i: (i, 0)),
        out_shape=jax.ShapeDtypeStruct((N, 16), jnp.float32),
    )(r2, h2p, degp, b2)


def _tc_loss(yp, yt2d):
    return pl.pallas_call(
        _tc_loss_body,
        out_shape=jax.ShapeDtypeStruct((1, 1), jnp.float32),
    )(yp, yt2d)


def kernel(x, edge_index, node_ids, label_inds, W1, b1, W2, b2):
    i32 = jnp.int32
    src = edge_index[0]
    dst = edge_index[1]
    fpad = TOTCH * CHUNK - E
    srcs = jnp.concatenate([src, jnp.zeros((fpad,), i32)]).reshape(TOTCH, CHUNK)
    dsts = jnp.concatenate([dst, jnp.full((fpad,), N, i32)]).reshape(TOTCH, CHUNK)
    fpad2 = TOTCH2 * CHUNK2 - E
    srcs2 = jnp.concatenate([src, jnp.zeros((fpad2,), i32)]).reshape(TOTCH2, CHUNK2)
    dsts2 = jnp.concatenate([dst, jnp.full((fpad2,), N, i32)]).reshape(TOTCH2, CHUNK2)
    z16 = jnp.zeros((NPAD, 16), jnp.float32)
    z128 = jnp.zeros((NPAD, 128), jnp.float32)
    ones16 = jnp.ones((CHUNK2, 16), jnp.float32)
    nid_p = jnp.concatenate([node_ids, jnp.zeros((NIDS_PAD - NIDS,), i32)])
    lab_p = jnp.concatenate([label_inds, jnp.zeros((NPAD - N,), i32)])

    degp = _sc_deg(dsts2, ones16, z16)
    h1p = _tc1(x, W1, degp)
    r1 = _sc_pass128(h1p, srcs, dsts, z128)
    h2p = _tc2(r1, h1p, degp, b1.reshape(1, 128), W2)
    r2 = _sc_pass16(h2p, srcs2, dsts2, z16)
    out2 = _tc3(r2, h2p, degp, b2.reshape(1, 16))
    yp, yt = _sc_pred(out2, nid_p, lab_p)
    loss = _tc_loss(yp, yt.reshape(16, 128))
    return (loss[0, 0], yp[:NIDS])


# speedup vs baseline: 1.0458x; 1.0458x over previous
import functools

import numpy as np
import jax
import jax.numpy as jnp
from jax import lax
from jax.experimental import pallas as pl
from jax.experimental.pallas import tpu as pltpu
from jax.experimental.pallas import tpu_sc as plsc

N = 10000
NPAD = 10112
ROWS_PER_TILE = NPAD // 16
E = 320000
NW = 32
CHUNK = 64
NCH = 160
CHUNK2 = 128
NCH2 = 80
NCH_S = 132
NCH_F = 182
NCH_PAIR = NCH_S + NCH_F
TOTCH = 16 * NCH_PAIR + NCH_F
NCH2_S = 68
NCH2_F = 90
NCH2_PAIR = NCH2_S + NCH2_F
TOTCH2 = 16 * NCH2_PAIR + NCH2_F
EPT = NCH * CHUNK
ETOT = EPT * NW
NIDS = 2000
NIDS_PAD = 2048
IDS_PER_TILE = NIDS_PAD // NW

_MESH = plsc.VectorSubcoreMesh(core_axis_name="c", subcore_axis_name="s")
_SC_PARAMS = pltpu.CompilerParams(use_tc_tiling_on_sc=False,
                                  needs_layout_passes=False)


def _tile_ids():
    c = lax.axis_index("c")
    s = lax.axis_index("s")
    return c, s, s * 2 + c


@functools.partial(
    pl.kernel,
    out_type=jax.ShapeDtypeStruct((2, NPAD, 16), jnp.float32),
    mesh=_MESH,
    scratch_types=[
        pltpu.VMEM((NCH2_F, CHUNK2), jnp.int32),
        pltpu.VMEM((CHUNK2, 16), jnp.float32),
        pltpu.VMEM_SHARED((NPAD, 16), jnp.float32),
        pltpu.SemaphoreType.DMA,
    ],
    compiler_params=_SC_PARAMS,
)
def _sc_deg(dsts_hbm, ones_hbm, zeros_hbm, out_hbm, didx, ones_v, acc, sem):
    c, s, wid = _tile_ids()
    nch = lax.select(c == 0, NCH2_S, NCH2_F)
    start = s * NCH2_PAIR + c * NCH2_S
    rows = pl.ds(s * ROWS_PER_TILE, ROWS_PER_TILE)
    pltpu.sync_copy(zeros_hbm.at[rows], acc.at[rows])
    pltpu.sync_copy(dsts_hbm.at[pl.ds(start, NCH2_F)], didx)
    pltpu.sync_copy(ones_hbm, ones_v)
    plsc.subcore_barrier()

    def fire(j, _):
        pltpu.async_copy(ones_v, acc.at[didx.at[j]], sem, add=True)
        return _

    lax.fori_loop(0, nch, fire, None)

    def drain(j, _):
        pltpu.make_async_copy(ones_v, acc.at[didx.at[j]], sem).wait()
        return _

    lax.fori_loop(0, nch, drain, None)
    plsc.subcore_barrier()
    pltpu.sync_copy(acc.at[rows], out_hbm.at[c, rows])


def _make_sc_pass(d, nch_by_core, chunk, pair):
    nch_max = max(nch_by_core)

    @functools.partial(
        pl.kernel,
        out_type=jax.ShapeDtypeStruct((2, NPAD, d), jnp.float32),
        mesh=_MESH,
        scratch_types=[
            pltpu.VMEM((nch_max, chunk), jnp.int32),
            pltpu.VMEM((nch_max, chunk), jnp.int32),
            pltpu.VMEM((2, chunk, d), jnp.float32),
            pltpu.VMEM_SHARED((NPAD, d), jnp.float32),
            pltpu.SemaphoreType.DMA,
            pltpu.SemaphoreType.DMA,
        ],
        compiler_params=_SC_PARAMS,
    )
    def _sc_pass(h_hbm, srcs_hbm, dsts_hbm, zeros_hbm, out_hbm,
                 sidx, didx, gbuf, acc, sem0, sem1):
        c, s, wid = _tile_ids()
        nch = lax.select(c == 0, nch_by_core[0], nch_by_core[1])
        start = s * pair + c * nch_by_core[0]
        pltpu.sync_copy(srcs_hbm.at[pl.ds(start, nch_max)], sidx)
        pltpu.sync_copy(dsts_hbm.at[pl.ds(start, nch_max)], didx)
        rows = pl.ds(s * ROWS_PER_TILE, ROWS_PER_TILE)
        pltpu.sync_copy(zeros_hbm.at[rows], acc.at[rows])
        plsc.subcore_barrier()

        gb0, gb1 = gbuf.at[0], gbuf.at[1]
        pltpu.async_copy(h_hbm.at[sidx.at[0]], gb0, sem0)
        pltpu.async_copy(h_hbm.at[sidx.at[1]], gb1, sem1)

        def body(j, _):
            pltpu.make_async_copy(h_hbm.at[sidx.at[2 * j]], gb0, sem0).wait()
            pltpu.sync_copy(gb0, acc.at[didx.at[2 * j]], add=True)
            pltpu.async_copy(h_hbm.at[sidx.at[2 * j + 2]], gb0, sem0)
            pltpu.make_async_copy(h_hbm.at[sidx.at[2 * j + 1]], gb1, sem1).wait()
            pltpu.sync_copy(gb1, acc.at[didx.at[2 * j + 1]], add=True)
            pltpu.async_copy(h_hbm.at[sidx.at[2 * j + 3]], gb1, sem1)
            return _

        lax.fori_loop(0, nch // 2 - 1, body, None)
        pltpu.make_async_copy(h_hbm.at[sidx.at[nch - 2]], gb0, sem0).wait()
        pltpu.sync_copy(gb0, acc.at[didx.at[nch - 2]], add=True)
        pltpu.make_async_copy(h_hbm.at[sidx.at[nch - 1]], gb1, sem1).wait()
        pltpu.sync_copy(gb1, acc.at[didx.at[nch - 1]], add=True)
        plsc.subcore_barrier()
        pltpu.sync_copy(acc.at[rows], out_hbm.at[c, rows])

    return _sc_pass


_sc_pass128 = _make_sc_pass(128, (NCH_S, NCH_F), CHUNK, NCH_PAIR)
_sc_pass16 = _make_sc_pass(16, (NCH2_S, NCH2_F), CHUNK2, NCH2_PAIR)


@functools.partial(
    pl.kernel,
    out_type=(jax.ShapeDtypeStruct((NIDS_PAD, 16), jnp.float32),
              jax.ShapeDtypeStruct((NIDS_PAD,), jnp.int32)),
    mesh=_MESH,
    scratch_types=[
        pltpu.VMEM((IDS_PER_TILE,), jnp.int32),
        pltpu.VMEM((IDS_PER_TILE, 16), jnp.float32),
        pltpu.VMEM((NPAD,), jnp.int32),
        pltpu.VMEM((IDS_PER_TILE,), jnp.int32),
        pltpu.SemaphoreType.DMA,
    ],
    compiler_params=_SC_PARAMS,
)
def _sc_pred(out2_hbm, nid_hbm, lab_hbm, yp_hbm, yt_hbm,
             nid_v, rows_v, lab_v, yt_v, sem):
    _, _, wid = _tile_ids()
    sl = pl.ds(wid * IDS_PER_TILE, IDS_PER_TILE)
    pltpu.sync_copy(nid_hbm.at[sl], nid_v)
    pltpu.sync_copy(lab_hbm, lab_v)
    pltpu.async_copy(out2_hbm.at[nid_v], rows_v, sem).wait()
    pltpu.sync_copy(rows_v, yp_hbm.at[sl])
    for k in range(IDS_PER_TILE // 16):
        idx = nid_v[pl.ds(k * 16, 16)]
        yt_v[pl.ds(k * 16, 16)] = plsc.load_gather(lab_v, [idx])
    pltpu.sync_copy(yt_v, yt_hbm.at[sl])


def _dinv_from_degp(degp_blk):
    deg = degp_blk[0, :, 0] + degp_blk[1, :, 0] + 1.0
    return lax.rsqrt(deg)


def _tc1_body(x_ref, w_ref, degp_ref, o_ref):
    dinv = _dinv_from_degp(degp_ref[...])
    h = jnp.dot(x_ref[...], w_ref[...], preferred_element_type=jnp.float32)
    o_ref[...] = h * dinv[:, None]


def _tc2_body(r1_ref, h1p_ref, degp_ref, b1_ref, w2_ref, o_ref):
    dinv = _dinv_from_degp(degp_ref[...])
    s = r1_ref[0] + r1_ref[1] + h1p_ref[...]
    h2 = jnp.maximum(s * dinv[:, None] + b1_ref[...], 0.0)
    o_ref[...] = jnp.dot(h2, w2_ref[...],
                         preferred_element_type=jnp.float32) * dinv[:, None]


def _tc3_body(r2_ref, h2p_ref, degp_ref, b2_ref, o_ref):
    dinv = _dinv_from_degp(degp_ref[...])
    s = r2_ref[0] + r2_ref[1] + h2p_ref[...]
    o_ref[...] = s * dinv[:, None] + b2_ref[...]


def _tc_loss_body(yp_ref, yt_ref, o_ref):
    yp = yp_ref[...].reshape(16, 128, 16)
    lab = yt_ref[...]
    m = jnp.max(yp, axis=2)
    lse = jnp.log(jnp.sum(jnp.exp(yp - m[:, :, None]), axis=2)) + m
    onehot = (lax.broadcasted_iota(jnp.int32, (16, 128, 16), 2)
              == lab[:, :, None])
    pick = jnp.sum(jnp.where(onehot, yp, 0.0), axis=2)
    ridx = (lax.broadcasted_iota(jnp.int32, (16, 128), 0) * 128
            + lax.broadcasted_iota(jnp.int32, (16, 128), 1))
    nll = jnp.where(ridx < NIDS, lse - pick, 0.0)
    o_ref[...] = (jnp.sum(nll) / NIDS).reshape(1, 1)


_RB = 1000


def _tc1(x, w1, degp):
    return pl.pallas_call(
        _tc1_body,
        grid=(N // _RB,),
        in_specs=[
            pl.BlockSpec((_RB, 128), lambda i: (i, 0)),
            pl.BlockSpec((128, 128), lambda i: (0, 0)),
            pl.BlockSpec((2, _RB, 16), lambda i: (0, i, 0)),
        ],
        out_specs=pl.BlockSpec((_RB, 128), lambda i: (i, 0)),
        out_shape=jax.ShapeDtypeStruct((N, 128), jnp.float32),
    )(x, w1, degp)


def _tc2(r1, h1p, degp, b1, w2):
    return pl.pallas_call(
        _tc2_body,
        grid=(N // _RB,),
        in_specs=[
            pl.BlockSpec((2, _RB, 128), lambda i: (0, i, 0)),
            pl.BlockSpec((_RB, 128), lambda i: (i, 0)),
            pl.BlockSpec((2, _RB, 16), lambda i: (0, i, 0)),
            pl.BlockSpec((1, 128), lambda i: (0, 0)),
            pl.BlockSpec((128, 16), lambda i: (0, 0)),
        ],
        out_specs=pl.BlockSpec((_RB, 16), lambda i: (i, 0)),
        out_shape=jax.ShapeDtypeStruct((N, 16), jnp.float32),
    )(r1, h1p, degp, b1, w2)


def _tc3(r2, h2p, degp, b2):
    return pl.pallas_call(
        _tc3_body,
        grid=(N // _RB,),
        in_specs=[
            pl.BlockSpec((2, _RB, 16), lambda i: (0, i, 0)),
            pl.BlockSpec((_RB, 16), lambda i: (i, 0)),
            pl.BlockSpec((2, _RB, 16), lambda i: (0, i, 0)),
            pl.BlockSpec((1, 16), lambda i: (0, 0)),
        ],
        out_specs=pl.BlockSpec((_RB, 16), lambda i: (i, 0)),
        out_shape=jax.ShapeDtypeStruct((N, 16), jnp.float32),
    )(r2, h2p, degp, b2)


def _tc_loss(yp, yt2d):
    return pl.pallas_call(
        _tc_loss_body,
        out_shape=jax.ShapeDtypeStruct((1, 1), jnp.float32),
    )(yp, yt2d)


def kernel(x, edge_index, node_ids, label_inds, W1, b1, W2, b2):
    i32 = jnp.int32
    src = edge_index[0]
    dst = edge_index[1]
    fpad = TOTCH * CHUNK - E
    srcs = jnp.concatenate([src, jnp.zeros((fpad,), i32)]).reshape(TOTCH, CHUNK)
    dsts = jnp.concatenate([dst, jnp.full((fpad,), N, i32)]).reshape(TOTCH, CHUNK)
    fpad2 = TOTCH2 * CHUNK2 - E
    srcs2 = jnp.concatenate([src, jnp.zeros((fpad2,), i32)]).reshape(TOTCH2, CHUNK2)
    dsts2 = jnp.concatenate([dst, jnp.full((fpad2,), N, i32)]).reshape(TOTCH2, CHUNK2)
    z16 = jnp.zeros((NPAD, 16), jnp.float32)
    z128 = jnp.zeros((NPAD, 128), jnp.float32)
    ones16 = jnp.ones((CHUNK2, 16), jnp.float32)
    nid_p = jnp.concatenate([node_ids, jnp.zeros((NIDS_PAD - NIDS,), i32)])
    lab_p = jnp.concatenate([label_inds, jnp.zeros((NPAD - N,), i32)])

    degp = _sc_deg(dsts2, ones16, z16)
    h1p = _tc1(x, W1, degp)
    r1 = _sc_pass128(h1p, srcs, dsts, z128)
    h2p = _tc2(r1, h1p, degp, b1.reshape(1, 128), W2)
    r2 = _sc_pass16(h2p, srcs2, dsts2, z16)
    out2 = _tc3(r2, h2p, degp, b2.reshape(1, 16))
    yp, yt = _sc_pred(out2, nid_p, lab_p)
    loss = _tc_loss(yp, yt.reshape(16, 128))
    return (loss[0, 0], yp[:NIDS])


# speedup vs baseline: 1.0707x; 1.0238x over previous
import functools

import numpy as np
import jax
import jax.numpy as jnp
from jax import lax
from jax.experimental import pallas as pl
from jax.experimental.pallas import tpu as pltpu
from jax.experimental.pallas import tpu_sc as plsc

N = 10000
NPAD = 10112
ROWS_PER_TILE = NPAD // 16
E = 320000
NW = 32
CHUNK = 64
NCH = 160
CHUNK2 = 128
NCH2 = 80
NCH_S = 146
NCH_F = 168
NCH_PAIR = NCH_S + NCH_F
TOTCH = 16 * NCH_PAIR + NCH_F
NCH2_S = 68
NCH2_F = 90
NCH2_PAIR = NCH2_S + NCH2_F
TOTCH2 = 16 * NCH2_PAIR + NCH2_F
EPT = NCH * CHUNK
ETOT = EPT * NW
NIDS = 2000
NIDS_PAD = 2048
IDS_PER_TILE = NIDS_PAD // NW

_MESH = plsc.VectorSubcoreMesh(core_axis_name="c", subcore_axis_name="s")
_SC_PARAMS = pltpu.CompilerParams(use_tc_tiling_on_sc=False,
                                  needs_layout_passes=False)


def _tile_ids():
    c = lax.axis_index("c")
    s = lax.axis_index("s")
    return c, s, s * 2 + c


@functools.partial(
    pl.kernel,
    out_type=jax.ShapeDtypeStruct((2, NPAD, 16), jnp.float32),
    mesh=_MESH,
    scratch_types=[
        pltpu.VMEM((NCH2_F, CHUNK2), jnp.int32),
        pltpu.VMEM((CHUNK2, 16), jnp.float32),
        pltpu.VMEM_SHARED((NPAD, 16), jnp.float32),
        pltpu.SemaphoreType.DMA,
    ],
    compiler_params=_SC_PARAMS,
)
def _sc_deg(dsts_hbm, ones_hbm, zeros_hbm, out_hbm, didx, ones_v, acc, sem):
    c, s, wid = _tile_ids()
    nch = lax.select(c == 0, NCH2_S, NCH2_F)
    start = s * NCH2_PAIR + c * NCH2_S
    rows = pl.ds(s * ROWS_PER_TILE, ROWS_PER_TILE)
    pltpu.sync_copy(zeros_hbm.at[rows], acc.at[rows])
    pltpu.sync_copy(dsts_hbm.at[pl.ds(start, NCH2_F)], didx)
    pltpu.sync_copy(ones_hbm, ones_v)
    plsc.subcore_barrier()

    def fire(j, _):
        pltpu.async_copy(ones_v, acc.at[didx.at[j]], sem, add=True)
        return _

    lax.fori_loop(0, nch, fire, None)

    def drain(j, _):
        pltpu.make_async_copy(ones_v, acc.at[didx.at[j]], sem).wait()
        return _

    lax.fori_loop(0, nch, drain, None)
    plsc.subcore_barrier()
    pltpu.sync_copy(acc.at[rows], out_hbm.at[c, rows])


def _make_sc_pass(d, nch_by_core, chunk, pair):
    nch_max = max(nch_by_core)

    @functools.partial(
        pl.kernel,
        out_type=jax.ShapeDtypeStruct((2, NPAD, d), jnp.float32),
        mesh=_MESH,
        scratch_types=[
            pltpu.VMEM((nch_max, chunk), jnp.int32),
            pltpu.VMEM((nch_max, chunk), jnp.int32),
            pltpu.VMEM((2, chunk, d), jnp.float32),
            pltpu.VMEM_SHARED((NPAD, d), jnp.float32),
            pltpu.SemaphoreType.DMA,
            pltpu.SemaphoreType.DMA,
        ],
        compiler_params=_SC_PARAMS,
    )
    def _sc_pass(h_hbm, srcs_hbm, dsts_hbm, zeros_hbm, out_hbm,
                 sidx, didx, gbuf, acc, sem0, sem1):
        c, s, wid = _tile_ids()
        nch = lax.select(c == 0, nch_by_core[0], nch_by_core[1])
        start = s * pair + c * nch_by_core[0]
        pltpu.sync_copy(srcs_hbm.at[pl.ds(start, nch_max)], sidx)
        pltpu.sync_copy(dsts_hbm.at[pl.ds(start, nch_max)], didx)
        rows = pl.ds(s * ROWS_PER_TILE, ROWS_PER_TILE)
        pltpu.sync_copy(zeros_hbm.at[rows], acc.at[rows])
        plsc.subcore_barrier()

        gb0, gb1 = gbuf.at[0], gbuf.at[1]
        pltpu.async_copy(h_hbm.at[sidx.at[0]], gb0, sem0)
        pltpu.async_copy(h_hbm.at[sidx.at[1]], gb1, sem1)

        def body(j, _):
            pltpu.make_async_copy(h_hbm.at[sidx.at[2 * j]], gb0, sem0).wait()
            pltpu.sync_copy(gb0, acc.at[didx.at[2 * j]], add=True)
            pltpu.async_copy(h_hbm.at[sidx.at[2 * j + 2]], gb0, sem0)
            pltpu.make_async_copy(h_hbm.at[sidx.at[2 * j + 1]], gb1, sem1).wait()
            pltpu.sync_copy(gb1, acc.at[didx.at[2 * j + 1]], add=True)
            pltpu.async_copy(h_hbm.at[sidx.at[2 * j + 3]], gb1, sem1)
            return _

        lax.fori_loop(0, nch // 2 - 1, body, None)
        pltpu.make_async_copy(h_hbm.at[sidx.at[nch - 2]], gb0, sem0).wait()
        pltpu.sync_copy(gb0, acc.at[didx.at[nch - 2]], add=True)
        pltpu.make_async_copy(h_hbm.at[sidx.at[nch - 1]], gb1, sem1).wait()
        pltpu.sync_copy(gb1, acc.at[didx.at[nch - 1]], add=True)
        plsc.subcore_barrier()
        pltpu.sync_copy(acc.at[rows], out_hbm.at[c, rows])

    return _sc_pass


_sc_pass128 = _make_sc_pass(128, (NCH_S, NCH_F), CHUNK, NCH_PAIR)
_sc_pass16 = _make_sc_pass(16, (NCH2_S, NCH2_F), CHUNK2, NCH2_PAIR)


@functools.partial(
    pl.kernel,
    out_type=(jax.ShapeDtypeStruct((NIDS_PAD, 16), jnp.float32),
              jax.ShapeDtypeStruct((NIDS_PAD,), jnp.int32)),
    mesh=_MESH,
    scratch_types=[
        pltpu.VMEM((IDS_PER_TILE,), jnp.int32),
        pltpu.VMEM((IDS_PER_TILE, 16), jnp.float32),
        pltpu.VMEM((NPAD,), jnp.int32),
        pltpu.VMEM((IDS_PER_TILE,), jnp.int32),
        pltpu.SemaphoreType.DMA,
    ],
    compiler_params=_SC_PARAMS,
)
def _sc_pred(out2_hbm, nid_hbm, lab_hbm, yp_hbm, yt_hbm,
             nid_v, rows_v, lab_v, yt_v, sem):
    _, _, wid = _tile_ids()
    sl = pl.ds(wid * IDS_PER_TILE, IDS_PER_TILE)
    pltpu.sync_copy(nid_hbm.at[sl], nid_v)
    pltpu.sync_copy(lab_hbm, lab_v)
    pltpu.async_copy(out2_hbm.at[nid_v], rows_v, sem).wait()
    pltpu.sync_copy(rows_v, yp_hbm.at[sl])
    for k in range(IDS_PER_TILE // 16):
        idx = nid_v[pl.ds(k * 16, 16)]
        yt_v[pl.ds(k * 16, 16)] = plsc.load_gather(lab_v, [idx])
    pltpu.sync_copy(yt_v, yt_hbm.at[sl])


def _dinv_from_degp(degp_blk):
    deg = degp_blk[0, :, 0] + degp_blk[1, :, 0] + 1.0
    return lax.rsqrt(deg)


def _tc1_body(x_ref, w_ref, degp_ref, o_ref):
    dinv = _dinv_from_degp(degp_ref[...])
    h = jnp.dot(x_ref[...], w_ref[...], preferred_element_type=jnp.float32)
    o_ref[...] = h * dinv[:, None]


def _tc2_body(r1_ref, h1p_ref, degp_ref, b1_ref, w2_ref, o_ref):
    dinv = _dinv_from_degp(degp_ref[...])
    s = r1_ref[0] + r1_ref[1] + h1p_ref[...]
    h2 = jnp.maximum(s * dinv[:, None] + b1_ref[...], 0.0)
    o_ref[...] = jnp.dot(h2, w2_ref[...],
                         preferred_element_type=jnp.float32) * dinv[:, None]


def _tc3_body(r2_ref, h2p_ref, degp_ref, b2_ref, o_ref):
    dinv = _dinv_from_degp(degp_ref[...])
    s = r2_ref[0] + r2_ref[1] + h2p_ref[...]
    o_ref[...] = s * dinv[:, None] + b2_ref[...]


def _tc_loss_body(yp_ref, yt_ref, o_ref):
    yp = yp_ref[...].reshape(16, 128, 16)
    lab = yt_ref[...]
    m = jnp.max(yp, axis=2)
    lse = jnp.log(jnp.sum(jnp.exp(yp - m[:, :, None]), axis=2)) + m
    onehot = (lax.broadcasted_iota(jnp.int32, (16, 128, 16), 2)
              == lab[:, :, None])
    pick = jnp.sum(jnp.where(onehot, yp, 0.0), axis=2)
    ridx = (lax.broadcasted_iota(jnp.int32, (16, 128), 0) * 128
            + lax.broadcasted_iota(jnp.int32, (16, 128), 1))
    nll = jnp.where(ridx < NIDS, lse - pick, 0.0)
    o_ref[...] = (jnp.sum(nll) / NIDS).reshape(1, 1)


_RB = 1000


def _tc1(x, w1, degp):
    return pl.pallas_call(
        _tc1_body,
        grid=(N // _RB,),
        in_specs=[
            pl.BlockSpec((_RB, 128), lambda i: (i, 0)),
            pl.BlockSpec((128, 128), lambda i: (0, 0)),
            pl.BlockSpec((2, _RB, 16), lambda i: (0, i, 0)),
        ],
        out_specs=pl.BlockSpec((_RB, 128), lambda i: (i, 0)),
        out_shape=jax.ShapeDtypeStruct((N, 128), jnp.float32),
    )(x, w1, degp)


def _tc2(r1, h1p, degp, b1, w2):
    return pl.pallas_call(
        _tc2_body,
        grid=(N // _RB,),
        in_specs=[
            pl.BlockSpec((2, _RB, 128), lambda i: (0, i, 0)),
            pl.BlockSpec((_RB, 128), lambda i: (i, 0)),
            pl.BlockSpec((2, _RB, 16), lambda i: (0, i, 0)),
            pl.BlockSpec((1, 128), lambda i: (0, 0)),
            pl.BlockSpec((128, 16), lambda i: (0, 0)),
        ],
        out_specs=pl.BlockSpec((_RB, 16), lambda i: (i, 0)),
        out_shape=jax.ShapeDtypeStruct((N, 16), jnp.float32),
    )(r1, h1p, degp, b1, w2)


def _tc3(r2, h2p, degp, b2):
    return pl.pallas_call(
        _tc3_body,
        grid=(N // _RB,),
        in_specs=[
            pl.BlockSpec((2, _RB, 16), lambda i: (0, i, 0)),
            pl.BlockSpec((_RB, 16), lambda i: (i, 0)),
            pl.BlockSpec((2, _RB, 16), lambda i: (0, i, 0)),
            pl.BlockSpec((1, 16), lambda i: (0, 0)),
        ],
        out_specs=pl.BlockSpec((_RB, 16), lambda i: (i, 0)),
        out_shape=jax.ShapeDtypeStruct((N, 16), jnp.float32),
    )(r2, h2p, degp, b2)


def _tc_loss(yp, yt2d):
    return pl.pallas_call(
        _tc_loss_body,
        out_shape=jax.ShapeDtypeStruct((1, 1), jnp.float32),
    )(yp, yt2d)


def kernel(x, edge_index, node_ids, label_inds, W1, b1, W2, b2):
    i32 = jnp.int32
    src = edge_index[0]
    dst = edge_index[1]
    fpad = TOTCH * CHUNK - E
    srcs = jnp.concatenate([src, jnp.zeros((fpad,), i32)]).reshape(TOTCH, CHUNK)
    dsts = jnp.concatenate([dst, jnp.full((fpad,), N, i32)]).reshape(TOTCH, CHUNK)
    fpad2 = TOTCH2 * CHUNK2 - E
    srcs2 = jnp.concatenate([src, jnp.zeros((fpad2,), i32)]).reshape(TOTCH2, CHUNK2)
    dsts2 = jnp.concatenate([dst, jnp.full((fpad2,), N, i32)]).reshape(TOTCH2, CHUNK2)
    z16 = jnp.zeros((NPAD, 16), jnp.float32)
    z128 = jnp.zeros((NPAD, 128), jnp.float32)
    ones16 = jnp.ones((CHUNK2, 16), jnp.float32)
    nid_p = jnp.concatenate([node_ids, jnp.zeros((NIDS_PAD - NIDS,), i32)])
    lab_p = jnp.concatenate([label_inds, jnp.zeros((NPAD - N,), i32)])

    degp = _sc_deg(dsts2, ones16, z16)
    h1p = _tc1(x, W1, degp)
    r1 = _sc_pass128(h1p, srcs, dsts, z128)
    h2p = _tc2(r1, h1p, degp, b1.reshape(1, 128), W2)
    r2 = _sc_pass16(h2p, srcs2, dsts2, z16)
    out2 = _tc3(r2, h2p, degp, b2.reshape(1, 16))
    yp, yt = _sc_pred(out2, nid_p, lab_p)
    loss = _tc_loss(yp, yt.reshape(16, 128))
    return (loss[0, 0], yp[:NIDS])


# speedup vs baseline: 1.1194x; 1.0455x over previous
import functools

import numpy as np
import jax
import jax.numpy as jnp
from jax import lax
from jax.experimental import pallas as pl
from jax.experimental.pallas import tpu as pltpu
from jax.experimental.pallas import tpu_sc as plsc

N = 10000
NPAD = 10112
ROWS_PER_TILE = NPAD // 16
E = 320000
NW = 32
CHUNK = 64
NCH = 160
CHUNK2 = 128
NCH2 = 80
NCH_S = 156
NCH_F = 158
NCH_PAIR = NCH_S + NCH_F
TOTCH = 16 * NCH_PAIR + NCH_F
NCH2_S = 68
NCH2_F = 90
NCH2_PAIR = NCH2_S + NCH2_F
TOTCH2 = 16 * NCH2_PAIR + NCH2_F
EPT = NCH * CHUNK
ETOT = EPT * NW
NIDS = 2000
NIDS_PAD = 2048
IDS_PER_TILE = NIDS_PAD // NW

_MESH = plsc.VectorSubcoreMesh(core_axis_name="c", subcore_axis_name="s")
_SC_PARAMS = pltpu.CompilerParams(use_tc_tiling_on_sc=False,
                                  needs_layout_passes=False)


def _tile_ids():
    c = lax.axis_index("c")
    s = lax.axis_index("s")
    return c, s, s * 2 + c


@functools.partial(
    pl.kernel,
    out_type=jax.ShapeDtypeStruct((2, NPAD, 16), jnp.float32),
    mesh=_MESH,
    scratch_types=[
        pltpu.VMEM((NCH2_F, CHUNK2), jnp.int32),
        pltpu.VMEM((CHUNK2, 16), jnp.float32),
        pltpu.VMEM_SHARED((NPAD, 16), jnp.float32),
        pltpu.SemaphoreType.DMA,
    ],
    compiler_params=_SC_PARAMS,
)
def _sc_deg(dsts_hbm, ones_hbm, zeros_hbm, out_hbm, didx, ones_v, acc, sem):
    c, s, wid = _tile_ids()
    nch = lax.select(c == 0, NCH2_S, NCH2_F)
    start = s * NCH2_PAIR + c * NCH2_S
    rows = pl.ds(s * ROWS_PER_TILE, ROWS_PER_TILE)
    pltpu.sync_copy(zeros_hbm.at[rows], acc.at[rows])
    pltpu.sync_copy(dsts_hbm.at[pl.ds(start, NCH2_F)], didx)
    pltpu.sync_copy(ones_hbm, ones_v)
    plsc.subcore_barrier()

    def fire(j, _):
        pltpu.async_copy(ones_v, acc.at[didx.at[j]], sem, add=True)
        return _

    lax.fori_loop(0, nch, fire, None)

    def drain(j, _):
        pltpu.make_async_copy(ones_v, acc.at[didx.at[j]], sem).wait()
        return _

    lax.fori_loop(0, nch, drain, None)
    plsc.subcore_barrier()
    pltpu.sync_copy(acc.at[rows], out_hbm.at[c, rows])


def _make_sc_pass(d, nch_by_core, chunk, pair):
    nch_max = max(nch_by_core)

    @functools.partial(
        pl.kernel,
        out_type=jax.ShapeDtypeStruct((2, NPAD, d), jnp.float32),
        mesh=_MESH,
        scratch_types=[
            pltpu.VMEM((nch_max, chunk), jnp.int32),
            pltpu.VMEM((nch_max, chunk), jnp.int32),
            pltpu.VMEM((2, chunk, d), jnp.float32),
            pltpu.VMEM_SHARED((NPAD, d), jnp.float32),
            pltpu.SemaphoreType.DMA,
            pltpu.SemaphoreType.DMA,
        ],
        compiler_params=_SC_PARAMS,
    )
    def _sc_pass(h_hbm, srcs_hbm, dsts_hbm, zeros_hbm, out_hbm,
                 sidx, didx, gbuf, acc, sem0, sem1):
        c, s, wid = _tile_ids()
        nch = lax.select(c == 0, nch_by_core[0], nch_by_core[1])
        start = s * pair + c * nch_by_core[0]
        pltpu.sync_copy(srcs_hbm.at[pl.ds(start, nch_max)], sidx)
        pltpu.sync_copy(dsts_hbm.at[pl.ds(start, nch_max)], didx)
        rows = pl.ds(s * ROWS_PER_TILE, ROWS_PER_TILE)
        pltpu.sync_copy(zeros_hbm.at[rows], acc.at[rows])
        plsc.subcore_barrier()

        gb0, gb1 = gbuf.at[0], gbuf.at[1]
        pltpu.async_copy(h_hbm.at[sidx.at[0]], gb0, sem0)
        pltpu.async_copy(h_hbm.at[sidx.at[1]], gb1, sem1)

        def body(j, _):
            pltpu.make_async_copy(h_hbm.at[sidx.at[2 * j]], gb0, sem0).wait()
            pltpu.sync_copy(gb0, acc.at[didx.at[2 * j]], add=True)
            pltpu.async_copy(h_hbm.at[sidx.at[2 * j + 2]], gb0, sem0)
            pltpu.make_async_copy(h_hbm.at[sidx.at[2 * j + 1]], gb1, sem1).wait()
            pltpu.sync_copy(gb1, acc.at[didx.at[2 * j + 1]], add=True)
            pltpu.async_copy(h_hbm.at[sidx.at[2 * j + 3]], gb1, sem1)
            return _

        lax.fori_loop(0, nch // 2 - 1, body, None)
        pltpu.make_async_copy(h_hbm.at[sidx.at[nch - 2]], gb0, sem0).wait()
        pltpu.sync_copy(gb0, acc.at[didx.at[nch - 2]], add=True)
        pltpu.make_async_copy(h_hbm.at[sidx.at[nch - 1]], gb1, sem1).wait()
        pltpu.sync_copy(gb1, acc.at[didx.at[nch - 1]], add=True)
        plsc.subcore_barrier()
        pltpu.sync_copy(acc.at[rows], out_hbm.at[c, rows])

    return _sc_pass


_sc_pass128 = _make_sc_pass(128, (NCH_S, NCH_F), CHUNK, NCH_PAIR)
_sc_pass16 = _make_sc_pass(16, (NCH2_S, NCH2_F), CHUNK2, NCH2_PAIR)


@functools.partial(
    pl.kernel,
    out_type=(jax.ShapeDtypeStruct((NIDS_PAD, 16), jnp.float32),
              jax.ShapeDtypeStruct((NIDS_PAD,), jnp.int32)),
    mesh=_MESH,
    scratch_types=[
        pltpu.VMEM((IDS_PER_TILE,), jnp.int32),
        pltpu.VMEM((IDS_PER_TILE, 16), jnp.float32),
        pltpu.VMEM((NPAD,), jnp.int32),
        pltpu.VMEM((IDS_PER_TILE,), jnp.int32),
        pltpu.SemaphoreType.DMA,
    ],
    compiler_params=_SC_PARAMS,
)
def _sc_pred(out2_hbm, nid_hbm, lab_hbm, yp_hbm, yt_hbm,
             nid_v, rows_v, lab_v, yt_v, sem):
    _, _, wid = _tile_ids()
    sl = pl.ds(wid * IDS_PER_TILE, IDS_PER_TILE)
    pltpu.sync_copy(nid_hbm.at[sl], nid_v)
    pltpu.sync_copy(lab_hbm, lab_v)
    pltpu.async_copy(out2_hbm.at[nid_v], rows_v, sem).wait()
    pltpu.sync_copy(rows_v, yp_hbm.at[sl])
    for k in range(IDS_PER_TILE // 16):
        idx = nid_v[pl.ds(k * 16, 16)]
        yt_v[pl.ds(k * 16, 16)] = plsc.load_gather(lab_v, [idx])
    pltpu.sync_copy(yt_v, yt_hbm.at[sl])


def _dinv_from_degp(degp_blk):
    deg = degp_blk[0, :, 0] + degp_blk[1, :, 0] + 1.0
    return lax.rsqrt(deg)


def _tc1_body(x_ref, w_ref, degp_ref, o_ref):
    dinv = _dinv_from_degp(degp_ref[...])
    h = jnp.dot(x_ref[...], w_ref[...], preferred_element_type=jnp.float32)
    o_ref[...] = h * dinv[:, None]


def _tc2_body(r1_ref, h1p_ref, degp_ref, b1_ref, w2_ref, o_ref):
    dinv = _dinv_from_degp(degp_ref[...])
    s = r1_ref[0] + r1_ref[1] + h1p_ref[...]
    h2 = jnp.maximum(s * dinv[:, None] + b1_ref[...], 0.0)
    o_ref[...] = jnp.dot(h2, w2_ref[...],
                         preferred_element_type=jnp.float32) * dinv[:, None]


def _tc3_body(r2_ref, h2p_ref, degp_ref, b2_ref, o_ref):
    dinv = _dinv_from_degp(degp_ref[...])
    s = r2_ref[0] + r2_ref[1] + h2p_ref[...]
    o_ref[...] = s * dinv[:, None] + b2_ref[...]


def _tc_loss_body(yp_ref, yt_ref, o_ref):
    yp = yp_ref[...].reshape(16, 128, 16)
    lab = yt_ref[...]
    m = jnp.max(yp, axis=2)
    lse = jnp.log(jnp.sum(jnp.exp(yp - m[:, :, None]), axis=2)) + m
    onehot = (lax.broadcasted_iota(jnp.int32, (16, 128, 16), 2)
              == lab[:, :, None])
    pick = jnp.sum(jnp.where(onehot, yp, 0.0), axis=2)
    ridx = (lax.broadcasted_iota(jnp.int32, (16, 128), 0) * 128
            + lax.broadcasted_iota(jnp.int32, (16, 128), 1))
    nll = jnp.where(ridx < NIDS, lse - pick, 0.0)
    o_ref[...] = (jnp.sum(nll) / NIDS).reshape(1, 1)


_RB = 1000


def _tc1(x, w1, degp):
    return pl.pallas_call(
        _tc1_body,
        grid=(N // _RB,),
        in_specs=[
            pl.BlockSpec((_RB, 128), lambda i: (i, 0)),
            pl.BlockSpec((128, 128), lambda i: (0, 0)),
            pl.BlockSpec((2, _RB, 16), lambda i: (0, i, 0)),
        ],
        out_specs=pl.BlockSpec((_RB, 128), lambda i: (i, 0)),
        out_shape=jax.ShapeDtypeStruct((N, 128), jnp.float32),
    )(x, w1, degp)


def _tc2(r1, h1p, degp, b1, w2):
    return pl.pallas_call(
        _tc2_body,
        grid=(N // _RB,),
        in_specs=[
            pl.BlockSpec((2, _RB, 128), lambda i: (0, i, 0)),
            pl.BlockSpec((_RB, 128), lambda i: (i, 0)),
            pl.BlockSpec((2, _RB, 16), lambda i: (0, i, 0)),
            pl.BlockSpec((1, 128), lambda i: (0, 0)),
            pl.BlockSpec((128, 16), lambda i: (0, 0)),
        ],
        out_specs=pl.BlockSpec((_RB, 16), lambda i: (i, 0)),
        out_shape=jax.ShapeDtypeStruct((N, 16), jnp.float32),
    )(r1, h1p, degp, b1, w2)


def _tc3(r2, h2p, degp, b2):
    return pl.pallas_call(
        _tc3_body,
        grid=(N // _RB,),
        in_specs=[
            pl.BlockSpec((2, _RB, 16), lambda i: (0, i, 0)),
            pl.BlockSpec((_RB, 16), lambda i: (i, 0)),
            pl.BlockSpec((2, _RB, 16), lambda i: (0, i, 0)),
            pl.BlockSpec((1, 16), lambda i: (0, 0)),
        ],
        out_specs=pl.BlockSpec((_RB, 16), lambda i: (i, 0)),
        out_shape=jax.ShapeDtypeStruct((N, 16), jnp.float32),
    )(r2, h2p, degp, b2)


def _tc_loss(yp, yt2d):
    return pl.pallas_call(
        _tc_loss_body,
        out_shape=jax.ShapeDtypeStruct((1, 1), jnp.float32),
    )(yp, yt2d)


def kernel(x, edge_index, node_ids, label_inds, W1, b1, W2, b2):
    i32 = jnp.int32
    src = edge_index[0]
    dst = edge_index[1]
    fpad = TOTCH * CHUNK - E
    srcs = jnp.concatenate([src, jnp.zeros((fpad,), i32)]).reshape(TOTCH, CHUNK)
    dsts = jnp.concatenate([dst, jnp.full((fpad,), N, i32)]).reshape(TOTCH, CHUNK)
    fpad2 = TOTCH2 * CHUNK2 - E
    srcs2 = jnp.concatenate([src, jnp.zeros((fpad2,), i32)]).reshape(TOTCH2, CHUNK2)
    dsts2 = jnp.concatenate([dst, jnp.full((fpad2,), N, i32)]).reshape(TOTCH2, CHUNK2)
    z16 = jnp.zeros((NPAD, 16), jnp.float32)
    z128 = jnp.zeros((NPAD, 128), jnp.float32)
    ones16 = jnp.ones((CHUNK2, 16), jnp.float32)
    nid_p = jnp.concatenate([node_ids, jnp.zeros((NIDS_PAD - NIDS,), i32)])
    lab_p = jnp.concatenate([label_inds, jnp.zeros((NPAD - N,), i32)])

    degp = _sc_deg(dsts2, ones16, z16)
    h1p = _tc1(x, W1, degp)
    r1 = _sc_pass128(h1p, srcs, dsts, z128)
    h2p = _tc2(r1, h1p, degp, b1.reshape(1, 128), W2)
    r2 = _sc_pass16(h2p, srcs2, dsts2, z16)
    out2 = _tc3(r2, h2p, degp, b2.reshape(1, 16))
    yp, yt = _sc_pred(out2, nid_p, lab_p)
    loss = _tc_loss(yp, yt.reshape(16, 128))
    return (loss[0, 0], yp[:NIDS])


# speedup vs baseline: 1.1323x; 1.0116x over previous
import functools

import numpy as np
import jax
import jax.numpy as jnp
from jax import lax
from jax.experimental import pallas as pl
from jax.experimental.pallas import tpu as pltpu
from jax.experimental.pallas import tpu_sc as plsc

N = 10000
NPAD = 10112
ROWS_PER_TILE = NPAD // 16
E = 320000
NW = 32
CHUNK = 64
NCH = 160
CHUNK2 = 128
NCH2 = 80
NCH_S = 156
NCH_F = 158
NCH_PAIR = NCH_S + NCH_F
TOTCH = 16 * NCH_PAIR + NCH_F
NCH2_S = 78
NCH2_F = 80
NCH2_PAIR = NCH2_S + NCH2_F
TOTCH2 = 16 * NCH2_PAIR + NCH2_F
EPT = NCH * CHUNK
ETOT = EPT * NW
NIDS = 2000
NIDS_PAD = 2048
IDS_PER_TILE = NIDS_PAD // NW

_MESH = plsc.VectorSubcoreMesh(core_axis_name="c", subcore_axis_name="s")
_SC_PARAMS = pltpu.CompilerParams(use_tc_tiling_on_sc=False,
                                  needs_layout_passes=False)


def _tile_ids():
    c = lax.axis_index("c")
    s = lax.axis_index("s")
    return c, s, s * 2 + c


@functools.partial(
    pl.kernel,
    out_type=jax.ShapeDtypeStruct((2, NPAD, 16), jnp.float32),
    mesh=_MESH,
    scratch_types=[
        pltpu.VMEM((NCH2_F, CHUNK2), jnp.int32),
        pltpu.VMEM((CHUNK2, 16), jnp.float32),
        pltpu.VMEM_SHARED((NPAD, 16), jnp.float32),
        pltpu.SemaphoreType.DMA,
    ],
    compiler_params=_SC_PARAMS,
)
def _sc_deg(dsts_hbm, ones_hbm, zeros_hbm, out_hbm, didx, ones_v, acc, sem):
    c, s, wid = _tile_ids()
    nch = lax.select(c == 0, NCH2_S, NCH2_F)
    start = s * NCH2_PAIR + c * NCH2_S
    rows = pl.ds(s * ROWS_PER_TILE, ROWS_PER_TILE)
    pltpu.sync_copy(zeros_hbm.at[rows], acc.at[rows])
    pltpu.sync_copy(dsts_hbm.at[pl.ds(start, NCH2_F)], didx)
    pltpu.sync_copy(ones_hbm, ones_v)
    plsc.subcore_barrier()

    def fire(j, _):
        pltpu.async_copy(ones_v, acc.at[didx.at[j]], sem, add=True)
        return _

    lax.fori_loop(0, nch, fire, None)

    def drain(j, _):
        pltpu.make_async_copy(ones_v, acc.at[didx.at[j]], sem).wait()
        return _

    lax.fori_loop(0, nch, drain, None)
    plsc.subcore_barrier()
    pltpu.sync_copy(acc.at[rows], out_hbm.at[c, rows])


def _make_sc_pass(d, nch_by_core, chunk, pair):
    nch_max = max(nch_by_core)

    @functools.partial(
        pl.kernel,
        out_type=jax.ShapeDtypeStruct((2, NPAD, d), jnp.float32),
        mesh=_MESH,
        scratch_types=[
            pltpu.VMEM((nch_max, chunk), jnp.int32),
            pltpu.VMEM((nch_max, chunk), jnp.int32),
            pltpu.VMEM((2, chunk, d), jnp.float32),
            pltpu.VMEM_SHARED((NPAD, d), jnp.float32),
            pltpu.SemaphoreType.DMA,
            pltpu.SemaphoreType.DMA,
        ],
        compiler_params=_SC_PARAMS,
    )
    def _sc_pass(h_hbm, srcs_hbm, dsts_hbm, zeros_hbm, out_hbm,
                 sidx, didx, gbuf, acc, sem0, sem1):
        c, s, wid = _tile_ids()
        nch = lax.select(c == 0, nch_by_core[0], nch_by_core[1])
        start = s * pair + c * nch_by_core[0]
        pltpu.sync_copy(srcs_hbm.at[pl.ds(start, nch_max)], sidx)
        pltpu.sync_copy(dsts_hbm.at[pl.ds(start, nch_max)], didx)
        rows = pl.ds(s * ROWS_PER_TILE, ROWS_PER_TILE)
        pltpu.sync_copy(zeros_hbm.at[rows], acc.at[rows])
        plsc.subcore_barrier()

        gb0, gb1 = gbuf.at[0], gbuf.at[1]
        pltpu.async_copy(h_hbm.at[sidx.at[0]], gb0, sem0)
        pltpu.async_copy(h_hbm.at[sidx.at[1]], gb1, sem1)

        def body(j, _):
            pltpu.make_async_copy(h_hbm.at[sidx.at[2 * j]], gb0, sem0).wait()
            pltpu.sync_copy(gb0, acc.at[didx.at[2 * j]], add=True)
            pltpu.async_copy(h_hbm.at[sidx.at[2 * j + 2]], gb0, sem0)
            pltpu.make_async_copy(h_hbm.at[sidx.at[2 * j + 1]], gb1, sem1).wait()
            pltpu.sync_copy(gb1, acc.at[didx.at[2 * j + 1]], add=True)
            pltpu.async_copy(h_hbm.at[sidx.at[2 * j + 3]], gb1, sem1)
            return _

        lax.fori_loop(0, nch // 2 - 1, body, None)
        pltpu.make_async_copy(h_hbm.at[sidx.at[nch - 2]], gb0, sem0).wait()
        pltpu.sync_copy(gb0, acc.at[didx.at[nch - 2]], add=True)
        pltpu.make_async_copy(h_hbm.at[sidx.at[nch - 1]], gb1, sem1).wait()
        pltpu.sync_copy(gb1, acc.at[didx.at[nch - 1]], add=True)
        plsc.subcore_barrier()
        pltpu.sync_copy(acc.at[rows], out_hbm.at[c, rows])

    return _sc_pass


_sc_pass128 = _make_sc_pass(128, (NCH_S, NCH_F), CHUNK, NCH_PAIR)
_sc_pass16 = _make_sc_pass(16, (NCH2_S, NCH2_F), CHUNK2, NCH2_PAIR)


@functools.partial(
    pl.kernel,
    out_type=(jax.ShapeDtypeStruct((NIDS_PAD, 16), jnp.float32),
              jax.ShapeDtypeStruct((NIDS_PAD,), jnp.int32)),
    mesh=_MESH,
    scratch_types=[
        pltpu.VMEM((IDS_PER_TILE,), jnp.int32),
        pltpu.VMEM((IDS_PER_TILE, 16), jnp.float32),
        pltpu.VMEM((NPAD,), jnp.int32),
        pltpu.VMEM((IDS_PER_TILE,), jnp.int32),
        pltpu.SemaphoreType.DMA,
    ],
    compiler_params=_SC_PARAMS,
)
def _sc_pred(out2_hbm, nid_hbm, lab_hbm, yp_hbm, yt_hbm,
             nid_v, rows_v, lab_v, yt_v, sem):
    _, _, wid = _tile_ids()
    sl = pl.ds(wid * IDS_PER_TILE, IDS_PER_TILE)
    pltpu.sync_copy(nid_hbm.at[sl], nid_v)
    pltpu.sync_copy(lab_hbm, lab_v)
    pltpu.async_copy(out2_hbm.at[nid_v], rows_v, sem).wait()
    pltpu.sync_copy(rows_v, yp_hbm.at[sl])
    for k in range(IDS_PER_TILE // 16):
        idx = nid_v[pl.ds(k * 16, 16)]
        yt_v[pl.ds(k * 16, 16)] = plsc.load_gather(lab_v, [idx])
    pltpu.sync_copy(yt_v, yt_hbm.at[sl])


def _dinv_from_degp(degp_blk):
    deg = degp_blk[0, :, 0] + degp_blk[1, :, 0] + 1.0
    return lax.rsqrt(deg)


def _tc1_body(x_ref, w_ref, degp_ref, o_ref):
    dinv = _dinv_from_degp(degp_ref[...])
    h = jnp.dot(x_ref[...], w_ref[...], preferred_element_type=jnp.float32)
    o_ref[...] = h * dinv[:, None]


def _tc2_body(r1_ref, h1p_ref, degp_ref, b1_ref, w2_ref, o_ref):
    dinv = _dinv_from_degp(degp_ref[...])
    s = r1_ref[0] + r1_ref[1] + h1p_ref[...]
    h2 = jnp.maximum(s * dinv[:, None] + b1_ref[...], 0.0)
    o_ref[...] = jnp.dot(h2, w2_ref[...],
                         preferred_element_type=jnp.float32) * dinv[:, None]


def _tc3_body(r2_ref, h2p_ref, degp_ref, b2_ref, o_ref):
    dinv = _dinv_from_degp(degp_ref[...])
    s = r2_ref[0] + r2_ref[1] + h2p_ref[...]
    o_ref[...] = s * dinv[:, None] + b2_ref[...]


def _tc_loss_body(yp_ref, yt_ref, o_ref):
    yp = yp_ref[...].reshape(16, 128, 16)
    lab = yt_ref[...]
    m = jnp.max(yp, axis=2)
    lse = jnp.log(jnp.sum(jnp.exp(yp - m[:, :, None]), axis=2)) + m
    onehot = (lax.broadcasted_iota(jnp.int32, (16, 128, 16), 2)
              == lab[:, :, None])
    pick = jnp.sum(jnp.where(onehot, yp, 0.0), axis=2)
    ridx = (lax.broadcasted_iota(jnp.int32, (16, 128), 0) * 128
            + lax.broadcasted_iota(jnp.int32, (16, 128), 1))
    nll = jnp.where(ridx < NIDS, lse - pick, 0.0)
    o_ref[...] = (jnp.sum(nll) / NIDS).reshape(1, 1)


_RB = 1000


def _tc1(x, w1, degp):
    return pl.pallas_call(
        _tc1_body,
        grid=(N // _RB,),
        in_specs=[
            pl.BlockSpec((_RB, 128), lambda i: (i, 0)),
            pl.BlockSpec((128, 128), lambda i: (0, 0)),
            pl.BlockSpec((2, _RB, 16), lambda i: (0, i, 0)),
        ],
        out_specs=pl.BlockSpec((_RB, 128), lambda i: (i, 0)),
        out_shape=jax.ShapeDtypeStruct((N, 128), jnp.float32),
    )(x, w1, degp)


def _tc2(r1, h1p, degp, b1, w2):
    return pl.pallas_call(
        _tc2_body,
        grid=(N // _RB,),
        in_specs=[
            pl.BlockSpec((2, _RB, 128), lambda i: (0, i, 0)),
            pl.BlockSpec((_RB, 128), lambda i: (i, 0)),
            pl.BlockSpec((2, _RB, 16), lambda i: (0, i, 0)),
            pl.BlockSpec((1, 128), lambda i: (0, 0)),
            pl.BlockSpec((128, 16), lambda i: (0, 0)),
        ],
        out_specs=pl.BlockSpec((_RB, 16), lambda i: (i, 0)),
        out_shape=jax.ShapeDtypeStruct((N, 16), jnp.float32),
    )(r1, h1p, degp, b1, w2)


def _tc3(r2, h2p, degp, b2):
    return pl.pallas_call(
        _tc3_body,
        grid=(N // _RB,),
        in_specs=[
            pl.BlockSpec((2, _RB, 16), lambda i: (0, i, 0)),
            pl.BlockSpec((_RB, 16), lambda i: (i, 0)),
            pl.BlockSpec((2, _RB, 16), lambda i: (0, i, 0)),
            pl.BlockSpec((1, 16), lambda i: (0, 0)),
        ],
        out_specs=pl.BlockSpec((_RB, 16), lambda i: (i, 0)),
        out_shape=jax.ShapeDtypeStruct((N, 16), jnp.float32),
    )(r2, h2p, degp, b2)


def _tc_loss(yp, yt2d):
    return pl.pallas_call(
        _tc_loss_body,
        out_shape=jax.ShapeDtypeStruct((1, 1), jnp.float32),
    )(yp, yt2d)


def kernel(x, edge_index, node_ids, label_inds, W1, b1, W2, b2):
    i32 = jnp.int32
    src = edge_index[0]
    dst = edge_index[1]
    fpad = TOTCH * CHUNK - E
    srcs = jnp.concatenate([src, jnp.zeros((fpad,), i32)]).reshape(TOTCH, CHUNK)
    dsts = jnp.concatenate([dst, jnp.full((fpad,), N, i32)]).reshape(TOTCH, CHUNK)
    fpad2 = TOTCH2 * CHUNK2 - E
    srcs2 = jnp.concatenate([src, jnp.zeros((fpad2,), i32)]).reshape(TOTCH2, CHUNK2)
    dsts2 = jnp.concatenate([dst, jnp.full((fpad2,), N, i32)]).reshape(TOTCH2, CHUNK2)
    z16 = jnp.zeros((NPAD, 16), jnp.float32)
    z128 = jnp.zeros((NPAD, 128), jnp.float32)
    ones16 = jnp.ones((CHUNK2, 16), jnp.float32)
    nid_p = jnp.concatenate([node_ids, jnp.zeros((NIDS_PAD - NIDS,), i32)])
    lab_p = jnp.concatenate([label_inds, jnp.zeros((NPAD - N,), i32)])

    degp = _sc_deg(dsts2, ones16, z16)
    h1p = _tc1(x, W1, degp)
    r1 = _sc_pass128(h1p, srcs, dsts, z128)
    h2p = _tc2(r1, h1p, degp, b1.reshape(1, 128), W2)
    r2 = _sc_pass16(h2p, srcs2, dsts2, z16)
    out2 = _tc3(r2, h2p, degp, b2.reshape(1, 16))
    yp, yt = _sc_pred(out2, nid_p, lab_p)
    loss = _tc_loss(yp, yt.reshape(16, 128))
    return (loss[0, 0], yp[:NIDS])
